# baseline (speedup 1.0000x reference)
import jax
import jax.numpy as jnp
from jax import lax
from jax.experimental import pallas as pl
from jax.experimental.pallas import tpu as pltpu
from jax.experimental.pallas import tpu_sc as plsc

N = 10000
E = 320000
D = 128
NC = 2
NS = 16
CH = 128
WCH = 80
CCH = 160
NSLACK = 240
NA = 10240
ZCH = 40
RPS = NA // NS
IB = 8


RD = 5
SUB = 64
SPB = IB * CH // SUB


def _agg_body(nchunks, ones_mode):
    nb = nchunks // IB

    def body(g_hbm, src_hbm, dst_hbm, out_hbm, *scr):
        idx_s, idx_d = scr[0], scr[1]
        rows = list(scr[2:2 + RD])
        rows0 = rows[0]
        acc = scr[2 + RD]
        semz, semw = scr[3 + RD], scr[4 + RD]
        semg = list(scr[5 + RD:5 + 2 * RD])
        sems = list(scr[5 + 2 * RD:7 + 2 * RD])
        c = lax.axis_index("c")
        s = lax.axis_index("s")

        for i in range(ZCH):
            for k in range(8):
                rows0[i, pl.ds(k * 16, 16)] = jnp.zeros((16,), jnp.float32)
        zbase = pl.multiple_of(s * RPS, 8)
        zd = [pltpu.async_copy(rows0.at[pl.ds(0, ZCH)],
                               acc.at[pl.ds(zbase + k * ZCH, ZCH)], semz)
              for k in range(RPS // ZCH)]
        for d in zd:
            d.wait()
        if ones_mode:
            pltpu.sync_copy(g_hbm.at[pl.ds(0, SUB)], rows0)
        plsc.subcore_barrier()

        def batch(t, carry):
            base = t * SPB
            pltpu.sync_copy(dst_hbm.at[c, s, pl.ds(base, SPB)], idx_d)
            if ones_mode:
                sd = [pltpu.async_copy(rows0.at[pl.ds(0, SUB)],
                                       acc.at[idx_d.at[j]], sems[0], add=True)
                      for j in range(SPB)]
                for d in sd:
                    d.wait()
                return carry
            pltpu.sync_copy(src_hbm.at[c, s, pl.ds(base, SPB)], idx_s)
            gd = [None] * SPB
            sd = [None] * SPB
            for step in range(SPB + RD - 1):
                gq = step
                sq = step - (RD - 1)
                if gq < SPB:
                    if gq >= RD:
                        sd[gq - RD].wait()
                    gd[gq] = pltpu.async_copy(
                        g_hbm.at[idx_s.at[gq]], rows[gq % RD], semg[gq % RD])
                if 0 <= sq < SPB:
                    gd[sq].wait()
                    sd[sq] = pltpu.async_copy(
                        rows[sq % RD], acc.at[idx_d.at[sq]], sems[sq % 2],
                        add=True)
            for q in range(SPB - RD, SPB):
                sd[q].wait()
            return carry

        lax.fori_loop(0, nb, batch, 0)
        plsc.subcore_barrier()

        wbase = pl.multiple_of(s * RPS, 8)
        nw_full = RPS // ZCH
        nw_last = (N - (NS - 1) * RPS) // ZCH

        @pl.when(s < NS - 1)
        def _():
            wd = [pltpu.async_copy(acc.at[pl.ds(wbase + k * ZCH, ZCH)],
                                   out_hbm.at[c, pl.ds(wbase + k * ZCH, ZCH)],
                                   semw)
                  for k in range(nw_full)]
            for d in wd:
                d.wait()

        @pl.when(s == NS - 1)
        def _():
            wd = [pltpu.async_copy(acc.at[pl.ds(wbase + k * ZCH, ZCH)],
                                   out_hbm.at[c, pl.ds(wbase + k * ZCH, ZCH)],
                                   semw)
                  for k in range(nw_last)]
            for d in wd:
                d.wait()
    return body


def _make_agg(nchunks, ones_mode=False):
    return pl.kernel(
        _agg_body(nchunks, ones_mode),
        out_type=jax.ShapeDtypeStruct((NC, N, D), jnp.float32),
        mesh=plsc.VectorSubcoreMesh(core_axis_name="c", subcore_axis_name="s"),
        scratch_types=(
            [pltpu.VMEM((SPB, SUB), jnp.int32)] * 2
            + [pltpu.VMEM((SUB, D), jnp.float32)] * RD
            + [pltpu.VMEM_SHARED((NA, D), jnp.float32)]
            + [pltpu.SemaphoreType.DMA] * (RD + 4)
        ),
    )


_agg_full = _make_agg(WCH)
_agg_branch = _make_agg(CCH)
_agg_deg = _make_agg(WCH, ones_mode=True)


R = 1000
G = N // R

_f32 = jnp.float32


def _mm0_body(degp_ref, x_ref, w_ref, dis_ref, g_ref):
    deg = degp_ref[0] + degp_ref[1] + 1.0
    dis = lax.rsqrt(deg)
    dis_ref[...] = dis
    g_ref[...] = jnp.dot(x_ref[...], w_ref[...],
                         preferred_element_type=_f32) * dis


def _mm_mid_body(s_ref, g_ref, dis_ref, b_ref, w_ref, out_ref):
    dis = dis_ref[...]
    x = jnp.maximum((s_ref[0] + s_ref[1] + g_ref[...]) * dis + b_ref[...], 0.0)
    out_ref[...] = jnp.dot(x, w_ref[...], preferred_element_type=_f32) * dis


def _mm_fork_body(s_ref, g_ref, dis_ref, b_ref, w3_ref, w5_ref, out_ref):
    dis = dis_ref[...]
    h = jnp.maximum((s_ref[0] + s_ref[1] + g_ref[...]) * dis + b_ref[...], 0.0)
    out_ref[0, :, :] = jnp.dot(h, w3_ref[...], preferred_element_type=_f32) * dis
    out_ref[1, :, :] = jnp.dot(h, w5_ref[...], preferred_element_type=_f32) * dis


def _mm_bmid_body(s_ref, g_ref, dis_ref, bb_ref, w4_ref, w6_ref, out_ref):
    dis = dis_ref[...]
    xm = jnp.maximum((s_ref[0] + g_ref[0]) * dis + bb_ref[0], 0.0)
    xs = jnp.maximum((s_ref[1] + g_ref[1]) * dis + bb_ref[1], 0.0)
    out_ref[0, :, :] = jnp.dot(xm, w4_ref[...], preferred_element_type=_f32) * dis
    out_ref[1, :, :] = jnp.dot(xs, w6_ref[...], preferred_element_type=_f32) * dis


def _mm_fin_body(s_ref, g_ref, dis_ref, bb_ref, mean_ref, std_ref):
    dis = dis_ref[...]
    mean_ref[...] = (s_ref[0] + g_ref[0]) * dis + bb_ref[0]
    std_ref[...] = (s_ref[1] + g_ref[1]) * dis + bb_ref[1]


def _spec2(index=lambda i: (0, i, 0)):
    return pl.BlockSpec((2, R, D), index)


_SPEC_ND = pl.BlockSpec((R, D), lambda i: (i, 0))
_SPEC_W = pl.BlockSpec((D, D), lambda i: (0, 0))
_SPEC_B = pl.BlockSpec((1, D), lambda i: (0, 0))
_SPEC_BB = pl.BlockSpec((2, 1, D), lambda i: (0, 0, 0))
_SHAPE_ND = jax.ShapeDtypeStruct((N, D), _f32)
_SHAPE_2ND = jax.ShapeDtypeStruct((2, N, D), _f32)

_mm0 = pl.pallas_call(
    _mm0_body, grid=(G,),
    in_specs=[_spec2(), _SPEC_ND, _SPEC_W],
    out_specs=[_SPEC_ND, _SPEC_ND],
    out_shape=[_SHAPE_ND, _SHAPE_ND],
)

_mm_mid = pl.pallas_call(
    _mm_mid_body, grid=(G,),
    in_specs=[_spec2(), _SPEC_ND, _SPEC_ND, _SPEC_B, _SPEC_W],
    out_specs=[_SPEC_ND],
    out_shape=[_SHAPE_ND],
)

_mm_fork = pl.pallas_call(
    _mm_fork_body, grid=(G,),
    in_specs=[_spec2(), _SPEC_ND, _SPEC_ND, _SPEC_B, _SPEC_W, _SPEC_W],
    out_specs=[_spec2()],
    out_shape=[_SHAPE_2ND],
)

_mm_bmid = pl.pallas_call(
    _mm_bmid_body, grid=(G,),
    in_specs=[_spec2(), _spec2(), _SPEC_ND, _SPEC_BB, _SPEC_W, _SPEC_W],
    out_specs=[_spec2()],
    out_shape=[_SHAPE_2ND],
)

_mm_fin = pl.pallas_call(
    _mm_fin_body, grid=(G,),
    in_specs=[_spec2(), _spec2(), _SPEC_ND, _SPEC_BB],
    out_specs=[_SPEC_ND, _SPEC_ND],
    out_shape=[_SHAPE_ND, _SHAPE_ND],
)


def kernel(X, adj, W0, b0, W1, b1, W2, b2, W3, b3, W4, b4, W5, b5, W6, b6):
    src = adj[0]
    dst = adj[1]
    padW = WCH * CH - E // (NC * NS)
    padC = CCH * CH - E // NS
    padsW = jnp.broadcast_to(N + jnp.arange(padW, dtype=jnp.int32) % NSLACK,
                             (NC * NS, padW))
    padsC = jnp.broadcast_to(N + jnp.arange(padC, dtype=jnp.int32) % NSLACK,
                             (NS, padC))
    srcA = jnp.pad(src.reshape(NC * NS, E // (NC * NS)),
                   ((0, 0), (0, padW))).reshape(NC, NS, WCH * CH // SUB, SUB)
    dstA = jnp.concatenate(
        [dst.reshape(NC * NS, E // (NC * NS)), padsW],
        axis=1).reshape(NC, NS, WCH * CH // SUB, SUB)
    src16 = jnp.pad(src.reshape(NS, E // NS),
                    ((0, 0), (0, padC))).reshape(NS, CCH * CH // SUB, SUB)
    dst16 = jnp.concatenate(
        [dst.reshape(NS, E // NS), padsC],
        axis=1).reshape(NS, CCH * CH // SUB, SUB)
    srcB = jnp.stack([src16, src16 + N])
    dstB = jnp.stack([dst16, dst16])
    onesCD = jnp.ones((CH, D), _f32)
    b0r, b1r, b2r = b0.reshape(1, D), b1.reshape(1, D), b2.reshape(1, D)
    bB35 = jnp.stack([b3.reshape(1, D), b5.reshape(1, D)])
    bB46 = jnp.stack([b4.reshape(1, D), b6.reshape(1, D)])

    srcAd = srcA + (jnp.arange(NC, dtype=jnp.int32) * N)[:, None, None, None]

    def dup(g):
        return jnp.concatenate([g, g])

    degp = _agg_deg(onesCD, srcA, dstA)
    dis, g0 = _mm0(degp, X, W0)
    S0 = _agg_full(dup(g0), srcAd, dstA)
    (g1,) = _mm_mid(S0, g0, dis, b0r, W1)
    S1 = _agg_full(dup(g1), srcAd, dstA)
    (g2,) = _mm_mid(S1, g1, dis, b1r, W2)
    S2 = _agg_full(dup(g2), srcAd, dstA)
    (gB,) = _mm_fork(S2, g2, dis, b2r, W3, W5)
    SB = _agg_branch(gB.reshape(2 * N, D), srcB, dstB)
    (gB2,) = _mm_bmid(SB, gB, dis, bB35, W4, W6)
    SB2 = _agg_branch(gB2.reshape(2 * N, D), srcB, dstB)
    mean, std = _mm_fin(SB2, gB2, dis, bB46)
    return (mean, std)


# speedup vs baseline: 1.0439x; 1.0439x over previous
import jax
import jax.numpy as jnp
from jax import lax
from jax.experimental import pallas as pl
from jax.experimental.pallas import tpu as pltpu
from jax.experimental.pallas import tpu_sc as plsc

N = 10000
E = 320000
D = 128
NC = 2
NS = 16
CH = 128
WCH = 80
CCH = 160
NSLACK = 240
NA = 10240
ZCH = 40
RPS = NA // NS
IB = 40


RD = 2
SUB = 128
SPB = IB * CH // SUB


def _agg_body(nchunks, ones_mode):
    nb = nchunks // IB

    def body(g_hbm, src_hbm, dst_hbm, out_hbm, *scr):
        idx_s, idx_d = scr[0], scr[1]
        rows = list(scr[2:2 + RD])
        rows0 = rows[0]
        acc = scr[2 + RD]
        semz, semw = scr[3 + RD], scr[4 + RD]
        semg = list(scr[5 + RD:5 + 2 * RD])
        sems = list(scr[5 + 2 * RD:7 + 2 * RD])
        c = lax.axis_index("c")
        s = lax.axis_index("s")

        for i in range(ZCH):
            for k in range(8):
                rows0[i, pl.ds(k * 16, 16)] = jnp.zeros((16,), jnp.float32)
        zbase = pl.multiple_of(s * RPS, 8)
        zd = [pltpu.async_copy(rows0.at[pl.ds(0, ZCH)],
                               acc.at[pl.ds(zbase + k * ZCH, ZCH)], semz)
              for k in range(RPS // ZCH)]
        for d in zd:
            d.wait()
        if ones_mode:
            pltpu.sync_copy(g_hbm.at[pl.ds(0, SUB)], rows0)
        plsc.subcore_barrier()

        def batch(t, carry):
            base = t * SPB
            pltpu.sync_copy(dst_hbm.at[c, s, pl.ds(base, SPB)], idx_d)
            if ones_mode:
                sd = [pltpu.async_copy(rows0.at[pl.ds(0, SUB)],
                                       acc.at[idx_d.at[j]], sems[0], add=True)
                      for j in range(SPB)]
                for d in sd:
                    d.wait()
                return carry
            pltpu.sync_copy(src_hbm.at[c, s, pl.ds(base, SPB)], idx_s)
            gd = [None] * SPB
            sd = [None] * SPB
            for step in range(SPB + RD - 1):
                gq = step
                sq = step - (RD - 1)
                if gq < SPB:
                    if gq >= RD:
                        sd[gq - RD].wait()
                    gd[gq] = pltpu.async_copy(
                        g_hbm.at[idx_s.at[gq]], rows[gq % RD], semg[gq % RD])
                if 0 <= sq < SPB:
                    gd[sq].wait()
                    sd[sq] = pltpu.async_copy(
                        rows[sq % RD], acc.at[idx_d.at[sq]], sems[sq % 2],
                        add=True)
            for q in range(SPB - RD, SPB):
                sd[q].wait()
            return carry

        lax.fori_loop(0, nb, batch, 0)
        plsc.subcore_barrier()

        wbase = pl.multiple_of(s * RPS, 8)
        nw_full = RPS // ZCH
        nw_last = (N - (NS - 1) * RPS) // ZCH

        @pl.when(s < NS - 1)
        def _():
            wd = [pltpu.async_copy(acc.at[pl.ds(wbase + k * ZCH, ZCH)],
                                   out_hbm.at[c, pl.ds(wbase + k * ZCH, ZCH)],
                                   semw)
                  for k in range(nw_full)]
            for d in wd:
                d.wait()

        @pl.when(s == NS - 1)
        def _():
            wd = [pltpu.async_copy(acc.at[pl.ds(wbase + k * ZCH, ZCH)],
                                   out_hbm.at[c, pl.ds(wbase + k * ZCH, ZCH)],
                                   semw)
                  for k in range(nw_last)]
            for d in wd:
                d.wait()
    return body


def _make_agg(nchunks, ones_mode=False):
    return pl.kernel(
        _agg_body(nchunks, ones_mode),
        out_type=jax.ShapeDtypeStruct((NC, N, D), jnp.float32),
        mesh=plsc.VectorSubcoreMesh(core_axis_name="c", subcore_axis_name="s"),
        scratch_types=(
            [pltpu.VMEM((SPB, SUB), jnp.int32)] * 2
            + [pltpu.VMEM((SUB, D), jnp.float32)] * RD
            + [pltpu.VMEM_SHARED((NA, D), jnp.float32)]
            + [pltpu.SemaphoreType.DMA] * (RD + 4)
        ),
    )


_agg_full = _make_agg(WCH)
_agg_branch = _make_agg(CCH)
_agg_deg = _make_agg(WCH, ones_mode=True)


R = 1000
G = N // R

_f32 = jnp.float32


def _mm0_body(degp_ref, x_ref, w_ref, dis_ref, g_ref):
    deg = degp_ref[0] + degp_ref[1] + 1.0
    dis = lax.rsqrt(deg)
    dis_ref[...] = dis
    g_ref[...] = jnp.dot(x_ref[...], w_ref[...],
                         preferred_element_type=_f32) * dis


def _mm_mid_body(s_ref, g_ref, dis_ref, b_ref, w_ref, out_ref):
    dis = dis_ref[...]
    x = jnp.maximum((s_ref[0] + s_ref[1] + g_ref[...]) * dis + b_ref[...], 0.0)
    out_ref[...] = jnp.dot(x, w_ref[...], preferred_element_type=_f32) * dis


def _mm_fork_body(s_ref, g_ref, dis_ref, b_ref, w3_ref, w5_ref, out_ref):
    dis = dis_ref[...]
    h = jnp.maximum((s_ref[0] + s_ref[1] + g_ref[...]) * dis + b_ref[...], 0.0)
    out_ref[0, :, :] = jnp.dot(h, w3_ref[...], preferred_element_type=_f32) * dis
    out_ref[1, :, :] = jnp.dot(h, w5_ref[...], preferred_element_type=_f32) * dis


def _mm_bmid_body(s_ref, g_ref, dis_ref, bb_ref, w4_ref, w6_ref, out_ref):
    dis = dis_ref[...]
    xm = jnp.maximum((s_ref[0] + g_ref[0]) * dis + bb_ref[0], 0.0)
    xs = jnp.maximum((s_ref[1] + g_ref[1]) * dis + bb_ref[1], 0.0)
    out_ref[0, :, :] = jnp.dot(xm, w4_ref[...], preferred_element_type=_f32) * dis
    out_ref[1, :, :] = jnp.dot(xs, w6_ref[...], preferred_element_type=_f32) * dis


def _mm_fin_body(s_ref, g_ref, dis_ref, bb_ref, mean_ref, std_ref):
    dis = dis_ref[...]
    mean_ref[...] = (s_ref[0] + g_ref[0]) * dis + bb_ref[0]
    std_ref[...] = (s_ref[1] + g_ref[1]) * dis + bb_ref[1]


def _spec2(index=lambda i: (0, i, 0)):
    return pl.BlockSpec((2, R, D), index)


_SPEC_ND = pl.BlockSpec((R, D), lambda i: (i, 0))
_SPEC_W = pl.BlockSpec((D, D), lambda i: (0, 0))
_SPEC_B = pl.BlockSpec((1, D), lambda i: (0, 0))
_SPEC_BB = pl.BlockSpec((2, 1, D), lambda i: (0, 0, 0))
_SHAPE_ND = jax.ShapeDtypeStruct((N, D), _f32)
_SHAPE_2ND = jax.ShapeDtypeStruct((2, N, D), _f32)

_mm0 = pl.pallas_call(
    _mm0_body, grid=(G,),
    in_specs=[_spec2(), _SPEC_ND, _SPEC_W],
    out_specs=[_SPEC_ND, _SPEC_ND],
    out_shape=[_SHAPE_ND, _SHAPE_ND],
)

_mm_mid = pl.pallas_call(
    _mm_mid_body, grid=(G,),
    in_specs=[_spec2(), _SPEC_ND, _SPEC_ND, _SPEC_B, _SPEC_W],
    out_specs=[_SPEC_ND],
    out_shape=[_SHAPE_ND],
)

_mm_fork = pl.pallas_call(
    _mm_fork_body, grid=(G,),
    in_specs=[_spec2(), _SPEC_ND, _SPEC_ND, _SPEC_B, _SPEC_W, _SPEC_W],
    out_specs=[_spec2()],
    out_shape=[_SHAPE_2ND],
)

_mm_bmid = pl.pallas_call(
    _mm_bmid_body, grid=(G,),
    in_specs=[_spec2(), _spec2(), _SPEC_ND, _SPEC_BB, _SPEC_W, _SPEC_W],
    out_specs=[_spec2()],
    out_shape=[_SHAPE_2ND],
)

_mm_fin = pl.pallas_call(
    _mm_fin_body, grid=(G,),
    in_specs=[_spec2(), _spec2(), _SPEC_ND, _SPEC_BB],
    out_specs=[_SPEC_ND, _SPEC_ND],
    out_shape=[_SHAPE_ND, _SHAPE_ND],
)


def kernel(X, adj, W0, b0, W1, b1, W2, b2, W3, b3, W4, b4, W5, b5, W6, b6):
    src = adj[0]
    dst = adj[1]
    padW = WCH * CH - E // (NC * NS)
    padC = CCH * CH - E // NS
    padsW = jnp.broadcast_to(N + jnp.arange(padW, dtype=jnp.int32) % NSLACK,
                             (NC * NS, padW))
    padsC = jnp.broadcast_to(N + jnp.arange(padC, dtype=jnp.int32) % NSLACK,
                             (NS, padC))
    srcA = jnp.pad(src.reshape(NC * NS, E // (NC * NS)),
                   ((0, 0), (0, padW))).reshape(NC, NS, WCH * CH // SUB, SUB)
    dstA = jnp.concatenate(
        [dst.reshape(NC * NS, E // (NC * NS)), padsW],
        axis=1).reshape(NC, NS, WCH * CH // SUB, SUB)
    src16 = jnp.pad(src.reshape(NS, E // NS),
                    ((0, 0), (0, padC))).reshape(NS, CCH * CH // SUB, SUB)
    dst16 = jnp.concatenate(
        [dst.reshape(NS, E // NS), padsC],
        axis=1).reshape(NS, CCH * CH // SUB, SUB)
    srcB = jnp.stack([src16, src16 + N])
    dstB = jnp.stack([dst16, dst16])
    onesCD = jnp.ones((CH, D), _f32)
    b0r, b1r, b2r = b0.reshape(1, D), b1.reshape(1, D), b2.reshape(1, D)
    bB35 = jnp.stack([b3.reshape(1, D), b5.reshape(1, D)])
    bB46 = jnp.stack([b4.reshape(1, D), b6.reshape(1, D)])

    srcAd = srcA + (jnp.arange(NC, dtype=jnp.int32) * N)[:, None, None, None]

    def dup(g):
        return jnp.concatenate([g, g])

    degp = _agg_deg(onesCD, srcA, dstA)
    dis, g0 = _mm0(degp, X, W0)
    S0 = _agg_full(dup(g0), srcAd, dstA)
    (g1,) = _mm_mid(S0, g0, dis, b0r, W1)
    S1 = _agg_full(dup(g1), srcAd, dstA)
    (g2,) = _mm_mid(S1, g1, dis, b1r, W2)
    S2 = _agg_full(dup(g2), srcAd, dstA)
    (gB,) = _mm_fork(S2, g2, dis, b2r, W3, W5)
    SB = _agg_branch(gB.reshape(2 * N, D), srcB, dstB)
    (gB2,) = _mm_bmid(SB, gB, dis, bB35, W4, W6)
    SB2 = _agg_branch(gB2.reshape(2 * N, D), srcB, dstB)
    mean, std = _mm_fin(SB2, gB2, dis, bB46)
    return (mean, std)


# speedup vs baseline: 1.0596x; 1.0150x over previous
import jax
import jax.numpy as jnp
from jax import lax
from jax.experimental import pallas as pl
from jax.experimental.pallas import tpu as pltpu
from jax.experimental.pallas import tpu_sc as plsc

N = 10000
E = 320000
D = 128
NC = 2
NS = 16
CH = 128
WCH = 80
CCH = 160
NSLACK = 240
NA = 10240
ZCH = 40
RPS = NA // NS
IB = 40


RD = 3
SUB = 64
SPB = IB * CH // SUB


def _agg_body(nchunks, ones_mode):
    nb = nchunks // IB

    def body(g_hbm, src_hbm, dst_hbm, out_hbm, *scr):
        idx_s, idx_d = scr[0], scr[1]
        rows = list(scr[2:2 + RD])
        rows0 = rows[0]
        acc = scr[2 + RD]
        semz, semw = scr[3 + RD], scr[4 + RD]
        semg = list(scr[5 + RD:5 + 2 * RD])
        sems = list(scr[5 + 2 * RD:7 + 2 * RD])
        c = lax.axis_index("c")
        s = lax.axis_index("s")

        for i in range(ZCH):
            for k in range(8):
                rows0[i, pl.ds(k * 16, 16)] = jnp.zeros((16,), jnp.float32)
        zbase = pl.multiple_of(s * RPS, 8)
        zd = [pltpu.async_copy(rows0.at[pl.ds(0, ZCH)],
                               acc.at[pl.ds(zbase + k * ZCH, ZCH)], semz)
              for k in range(RPS // ZCH)]
        for d in zd:
            d.wait()
        if ones_mode:
            pltpu.sync_copy(g_hbm.at[pl.ds(0, SUB)], rows0)
        plsc.subcore_barrier()

        def batch(t, carry):
            base = t * SPB
            pltpu.sync_copy(dst_hbm.at[c, s, pl.ds(base, SPB)], idx_d)
            if ones_mode:
                sd = [pltpu.async_copy(rows0.at[pl.ds(0, SUB)],
                                       acc.at[idx_d.at[j]], sems[0], add=True)
                      for j in range(SPB)]
                for d in sd:
                    d.wait()
                return carry
            pltpu.sync_copy(src_hbm.at[c, s, pl.ds(base, SPB)], idx_s)
            gd = [None] * SPB
            sd = [None] * SPB
            for step in range(SPB + RD - 1):
                gq = step
                sq = step - (RD - 1)
                if gq < SPB:
                    if gq >= RD:
                        sd[gq - RD].wait()
                    gd[gq] = pltpu.async_copy(
                        g_hbm.at[idx_s.at[gq]], rows[gq % RD], semg[gq % RD])
                if 0 <= sq < SPB:
                    gd[sq].wait()
                    sd[sq] = pltpu.async_copy(
                        rows[sq % RD], acc.at[idx_d.at[sq]], sems[sq % 2],
                        add=True)
            for q in range(SPB - RD, SPB):
                sd[q].wait()
            return carry

        lax.fori_loop(0, nb, batch, 0)
        plsc.subcore_barrier()

        wbase = pl.multiple_of(s * RPS, 8)
        nw_full = RPS // ZCH
        nw_last = (N - (NS - 1) * RPS) // ZCH

        @pl.when(s < NS - 1)
        def _():
            wd = [pltpu.async_copy(acc.at[pl.ds(wbase + k * ZCH, ZCH)],
                                   out_hbm.at[c, pl.ds(wbase + k * ZCH, ZCH)],
                                   semw)
                  for k in range(nw_full)]
            for d in wd:
                d.wait()

        @pl.when(s == NS - 1)
        def _():
            wd = [pltpu.async_copy(acc.at[pl.ds(wbase + k * ZCH, ZCH)],
                                   out_hbm.at[c, pl.ds(wbase + k * ZCH, ZCH)],
                                   semw)
                  for k in range(nw_last)]
            for d in wd:
                d.wait()
    return body


def _make_agg(nchunks, ones_mode=False):
    return pl.kernel(
        _agg_body(nchunks, ones_mode),
        out_type=jax.ShapeDtypeStruct((NC, N, D), jnp.float32),
        mesh=plsc.VectorSubcoreMesh(core_axis_name="c", subcore_axis_name="s"),
        scratch_types=(
            [pltpu.VMEM((SPB, SUB), jnp.int32)] * 2
            + [pltpu.VMEM((SUB, D), jnp.float32)] * RD
            + [pltpu.VMEM_SHARED((NA, D), jnp.float32)]
            + [pltpu.SemaphoreType.DMA] * (RD + 4)
        ),
    )


_agg_full = _make_agg(WCH)
_agg_branch = _make_agg(CCH)
_agg_deg = _make_agg(WCH, ones_mode=True)


R = 1000
G = N // R

_f32 = jnp.float32


def _mm0_body(degp_ref, x_ref, w_ref, dis_ref, g_ref):
    deg = degp_ref[0] + degp_ref[1] + 1.0
    dis = lax.rsqrt(deg)
    dis_ref[...] = dis
    g_ref[...] = jnp.dot(x_ref[...], w_ref[...],
                         preferred_element_type=_f32) * dis


def _mm_mid_body(s_ref, g_ref, dis_ref, b_ref, w_ref, out_ref):
    dis = dis_ref[...]
    x = jnp.maximum((s_ref[0] + s_ref[1] + g_ref[...]) * dis + b_ref[...], 0.0)
    out_ref[...] = jnp.dot(x, w_ref[...], preferred_element_type=_f32) * dis


def _mm_fork_body(s_ref, g_ref, dis_ref, b_ref, w3_ref, w5_ref, out_ref):
    dis = dis_ref[...]
    h = jnp.maximum((s_ref[0] + s_ref[1] + g_ref[...]) * dis + b_ref[...], 0.0)
    out_ref[0, :, :] = jnp.dot(h, w3_ref[...], preferred_element_type=_f32) * dis
    out_ref[1, :, :] = jnp.dot(h, w5_ref[...], preferred_element_type=_f32) * dis


def _mm_bmid_body(s_ref, g_ref, dis_ref, bb_ref, w4_ref, w6_ref, out_ref):
    dis = dis_ref[...]
    xm = jnp.maximum((s_ref[0] + g_ref[0]) * dis + bb_ref[0], 0.0)
    xs = jnp.maximum((s_ref[1] + g_ref[1]) * dis + bb_ref[1], 0.0)
    out_ref[0, :, :] = jnp.dot(xm, w4_ref[...], preferred_element_type=_f32) * dis
    out_ref[1, :, :] = jnp.dot(xs, w6_ref[...], preferred_element_type=_f32) * dis


def _mm_fin_body(s_ref, g_ref, dis_ref, bb_ref, mean_ref, std_ref):
    dis = dis_ref[...]
    mean_ref[...] = (s_ref[0] + g_ref[0]) * dis + bb_ref[0]
    std_ref[...] = (s_ref[1] + g_ref[1]) * dis + bb_ref[1]


def _spec2(index=lambda i: (0, i, 0)):
    return pl.BlockSpec((2, R, D), index)


_SPEC_ND = pl.BlockSpec((R, D), lambda i: (i, 0))
_SPEC_W = pl.BlockSpec((D, D), lambda i: (0, 0))
_SPEC_B = pl.BlockSpec((1, D), lambda i: (0, 0))
_SPEC_BB = pl.BlockSpec((2, 1, D), lambda i: (0, 0, 0))
_SHAPE_ND = jax.ShapeDtypeStruct((N, D), _f32)
_SHAPE_2ND = jax.ShapeDtypeStruct((2, N, D), _f32)

_mm0 = pl.pallas_call(
    _mm0_body, grid=(G,),
    in_specs=[_spec2(), _SPEC_ND, _SPEC_W],
    out_specs=[_SPEC_ND, _SPEC_ND],
    out_shape=[_SHAPE_ND, _SHAPE_ND],
)

_mm_mid = pl.pallas_call(
    _mm_mid_body, grid=(G,),
    in_specs=[_spec2(), _SPEC_ND, _SPEC_ND, _SPEC_B, _SPEC_W],
    out_specs=[_SPEC_ND],
    out_shape=[_SHAPE_ND],
)

_mm_fork = pl.pallas_call(
    _mm_fork_body, grid=(G,),
    in_specs=[_spec2(), _SPEC_ND, _SPEC_ND, _SPEC_B, _SPEC_W, _SPEC_W],
    out_specs=[_spec2()],
    out_shape=[_SHAPE_2ND],
)

_mm_bmid = pl.pallas_call(
    _mm_bmid_body, grid=(G,),
    in_specs=[_spec2(), _spec2(), _SPEC_ND, _SPEC_BB, _SPEC_W, _SPEC_W],
    out_specs=[_spec2()],
    out_shape=[_SHAPE_2ND],
)

_mm_fin = pl.pallas_call(
    _mm_fin_body, grid=(G,),
    in_specs=[_spec2(), _spec2(), _SPEC_ND, _SPEC_BB],
    out_specs=[_SPEC_ND, _SPEC_ND],
    out_shape=[_SHAPE_ND, _SHAPE_ND],
)


def kernel(X, adj, W0, b0, W1, b1, W2, b2, W3, b3, W4, b4, W5, b5, W6, b6):
    src = adj[0]
    dst = adj[1]
    padW = WCH * CH - E // (NC * NS)
    padC = CCH * CH - E // NS
    padsW = jnp.broadcast_to(N + jnp.arange(padW, dtype=jnp.int32) % NSLACK,
                             (NC * NS, padW))
    padsC = jnp.broadcast_to(N + jnp.arange(padC, dtype=jnp.int32) % NSLACK,
                             (NS, padC))
    srcA = jnp.pad(src.reshape(NC * NS, E // (NC * NS)),
                   ((0, 0), (0, padW))).reshape(NC, NS, WCH * CH // SUB, SUB)
    dstA = jnp.concatenate(
        [dst.reshape(NC * NS, E // (NC * NS)), padsW],
        axis=1).reshape(NC, NS, WCH * CH // SUB, SUB)
    src16 = jnp.pad(src.reshape(NS, E // NS),
                    ((0, 0), (0, padC))).reshape(NS, CCH * CH // SUB, SUB)
    dst16 = jnp.concatenate(
        [dst.reshape(NS, E // NS), padsC],
        axis=1).reshape(NS, CCH * CH // SUB, SUB)
    srcB = jnp.stack([src16, src16 + N])
    dstB = jnp.stack([dst16, dst16])
    onesCD = jnp.ones((CH, D), _f32)
    b0r, b1r, b2r = b0.reshape(1, D), b1.reshape(1, D), b2.reshape(1, D)
    bB35 = jnp.stack([b3.reshape(1, D), b5.reshape(1, D)])
    bB46 = jnp.stack([b4.reshape(1, D), b6.reshape(1, D)])

    srcAd = srcA + (jnp.arange(NC, dtype=jnp.int32) * N)[:, None, None, None]

    def dup(g):
        return jnp.concatenate([g, g])

    degp = _agg_deg(onesCD, srcA, dstA)
    dis, g0 = _mm0(degp, X, W0)
    S0 = _agg_full(dup(g0), srcAd, dstA)
    (g1,) = _mm_mid(S0, g0, dis, b0r, W1)
    S1 = _agg_full(dup(g1), srcAd, dstA)
    (g2,) = _mm_mid(S1, g1, dis, b1r, W2)
    S2 = _agg_full(dup(g2), srcAd, dstA)
    (gB,) = _mm_fork(S2, g2, dis, b2r, W3, W5)
    SB = _agg_branch(gB.reshape(2 * N, D), srcB, dstB)
    (gB2,) = _mm_bmid(SB, gB, dis, bB35, W4, W6)
    SB2 = _agg_branch(gB2.reshape(2 * N, D), srcB, dstB)
    mean, std = _mm_fin(SB2, gB2, dis, bB46)
    return (mean, std)


# speedup vs baseline: 1.0660x; 1.0061x over previous
import jax
import jax.numpy as jnp
from jax import lax
from jax.experimental import pallas as pl
from jax.experimental.pallas import tpu as pltpu
from jax.experimental.pallas import tpu_sc as plsc

N = 10000
E = 320000
D = 128
NC = 2
NS = 16
CH = 128
WCH = 80
CCH = 160
NSLACK = 240
NA = 10240
ZCH = 40
RPS = NA // NS
IB = 40


RD = 3
SUB = 64
SPB = IB * CH // SUB


def _agg_body(nchunks, ones_mode):
    nb = nchunks // IB

    def body(g_hbm, src_hbm, dst_hbm, out_hbm, *scr):
        idx_s, idx_d = scr[0], scr[1]
        rows = list(scr[2:2 + RD])
        rows0 = rows[0]
        acc = scr[2 + RD]
        semz, semw = scr[3 + RD], scr[4 + RD]
        semg = list(scr[5 + RD:5 + 2 * RD])
        sems = list(scr[5 + 2 * RD:7 + 2 * RD])
        c = lax.axis_index("c")
        s = lax.axis_index("s")

        for i in range(ZCH):
            for k in range(8):
                rows0[i, pl.ds(k * 16, 16)] = jnp.zeros((16,), jnp.float32)
        zbase = pl.multiple_of(s * RPS, 8)
        zd = [pltpu.async_copy(rows0.at[pl.ds(0, ZCH)],
                               acc.at[pl.ds(zbase + k * ZCH, ZCH)], semz)
              for k in range(RPS // ZCH)]
        for d in zd:
            d.wait()
        if ones_mode:
            pltpu.sync_copy(g_hbm.at[pl.ds(0, SUB)], rows0)
        plsc.subcore_barrier()

        def batch(t, carry):
            base = t * SPB
            pltpu.sync_copy(dst_hbm.at[c, s, pl.ds(base, SPB)], idx_d)
            if ones_mode:
                sd = [pltpu.async_copy(rows0.at[pl.ds(0, SUB)],
                                       acc.at[idx_d.at[j]], sems[0], add=True)
                      for j in range(SPB)]
                for d in sd:
                    d.wait()
                return carry
            pltpu.sync_copy(src_hbm.at[c, s, pl.ds(base, SPB)], idx_s)
            gd = [None] * SPB
            sd = [None] * SPB
            for step in range(SPB + RD - 1):
                gq = step
                sq = step - (RD - 1)
                if gq < SPB:
                    if gq >= RD:
                        sd[gq - RD].wait()
                    gd[gq] = pltpu.async_copy(
                        g_hbm.at[idx_s.at[gq]], rows[gq % RD], semg[gq % RD])
                if 0 <= sq < SPB:
                    gd[sq].wait()
                    sd[sq] = pltpu.async_copy(
                        rows[sq % RD], acc.at[idx_d.at[sq]], sems[sq % 2],
                        add=True)
            for q in range(SPB - RD, SPB):
                sd[q].wait()
            return carry

        lax.fori_loop(0, nb, batch, 0)
        plsc.subcore_barrier()

        wbase = pl.multiple_of(s * RPS, 8)
        nw_full = RPS // ZCH
        nw_last = (N - (NS - 1) * RPS) // ZCH

        @pl.when(s < NS - 1)
        def _():
            wd = [pltpu.async_copy(acc.at[pl.ds(wbase + k * ZCH, ZCH)],
                                   out_hbm.at[c, pl.ds(wbase + k * ZCH, ZCH)],
                                   semw)
                  for k in range(nw_full)]
            for d in wd:
                d.wait()

        @pl.when(s == NS - 1)
        def _():
            wd = [pltpu.async_copy(acc.at[pl.ds(wbase + k * ZCH, ZCH)],
                                   out_hbm.at[c, pl.ds(wbase + k * ZCH, ZCH)],
                                   semw)
                  for k in range(nw_last)]
            for d in wd:
                d.wait()
    return body


def _make_agg(nchunks, ones_mode=False):
    return pl.kernel(
        _agg_body(nchunks, ones_mode),
        out_type=jax.ShapeDtypeStruct((NC, N, D), jnp.float32),
        mesh=plsc.VectorSubcoreMesh(core_axis_name="c", subcore_axis_name="s"),
        scratch_types=(
            [pltpu.VMEM((SPB, SUB), jnp.int32)] * 2
            + [pltpu.VMEM((SUB, D), jnp.float32)] * RD
            + [pltpu.VMEM_SHARED((NA, D), jnp.float32)]
            + [pltpu.SemaphoreType.DMA] * (RD + 4)
        ),
    )


_agg_full = _make_agg(WCH)
_agg_branch = _make_agg(CCH)
_agg_deg = _make_agg(WCH, ones_mode=True)


R = 2000
G = N // R

_f32 = jnp.float32


def _mm0_body(degp_ref, x_ref, w_ref, dis_ref, g_ref):
    deg = degp_ref[0] + degp_ref[1] + 1.0
    dis = lax.rsqrt(deg)
    dis_ref[...] = dis
    g_ref[...] = jnp.dot(x_ref[...], w_ref[...],
                         preferred_element_type=_f32) * dis


def _mm_mid_body(s_ref, g_ref, dis_ref, b_ref, w_ref, out_ref):
    dis = dis_ref[...]
    x = jnp.maximum((s_ref[0] + s_ref[1] + g_ref[...]) * dis + b_ref[...], 0.0)
    out_ref[...] = jnp.dot(x, w_ref[...], preferred_element_type=_f32) * dis


def _mm_fork_body(s_ref, g_ref, dis_ref, b_ref, w3_ref, w5_ref, out_ref):
    dis = dis_ref[...]
    h = jnp.maximum((s_ref[0] + s_ref[1] + g_ref[...]) * dis + b_ref[...], 0.0)
    out_ref[0, :, :] = jnp.dot(h, w3_ref[...], preferred_element_type=_f32) * dis
    out_ref[1, :, :] = jnp.dot(h, w5_ref[...], preferred_element_type=_f32) * dis


def _mm_bmid_body(s_ref, g_ref, dis_ref, bb_ref, w4_ref, w6_ref, out_ref):
    dis = dis_ref[...]
    xm = jnp.maximum((s_ref[0] + g_ref[0]) * dis + bb_ref[0], 0.0)
    xs = jnp.maximum((s_ref[1] + g_ref[1]) * dis + bb_ref[1], 0.0)
    out_ref[0, :, :] = jnp.dot(xm, w4_ref[...], preferred_element_type=_f32) * dis
    out_ref[1, :, :] = jnp.dot(xs, w6_ref[...], preferred_element_type=_f32) * dis


def _mm_fin_body(s_ref, g_ref, dis_ref, bb_ref, mean_ref, std_ref):
    dis = dis_ref[...]
    mean_ref[...] = (s_ref[0] + g_ref[0]) * dis + bb_ref[0]
    std_ref[...] = (s_ref[1] + g_ref[1]) * dis + bb_ref[1]


def _spec2(index=lambda i: (0, i, 0)):
    return pl.BlockSpec((2, R, D), index)


_SPEC_ND = pl.BlockSpec((R, D), lambda i: (i, 0))
_SPEC_W = pl.BlockSpec((D, D), lambda i: (0, 0))
_SPEC_B = pl.BlockSpec((1, D), lambda i: (0, 0))
_SPEC_BB = pl.BlockSpec((2, 1, D), lambda i: (0, 0, 0))
_SHAPE_ND = jax.ShapeDtypeStruct((N, D), _f32)
_SHAPE_2ND = jax.ShapeDtypeStruct((2, N, D), _f32)

_mm0 = pl.pallas_call(
    _mm0_body, grid=(G,),
    in_specs=[_spec2(), _SPEC_ND, _SPEC_W],
    out_specs=[_SPEC_ND, _SPEC_ND],
    out_shape=[_SHAPE_ND, _SHAPE_ND],
)

_mm_mid = pl.pallas_call(
    _mm_mid_body, grid=(G,),
    in_specs=[_spec2(), _SPEC_ND, _SPEC_ND, _SPEC_B, _SPEC_W],
    out_specs=[_SPEC_ND],
    out_shape=[_SHAPE_ND],
)

_mm_fork = pl.pallas_call(
    _mm_fork_body, grid=(G,),
    in_specs=[_spec2(), _SPEC_ND, _SPEC_ND, _SPEC_B, _SPEC_W, _SPEC_W],
    out_specs=[_spec2()],
    out_shape=[_SHAPE_2ND],
)

_mm_bmid = pl.pallas_call(
    _mm_bmid_body, grid=(G,),
    in_specs=[_spec2(), _spec2(), _SPEC_ND, _SPEC_BB, _SPEC_W, _SPEC_W],
    out_specs=[_spec2()],
    out_shape=[_SHAPE_2ND],
)

_mm_fin = pl.pallas_call(
    _mm_fin_body, grid=(G,),
    in_specs=[_spec2(), _spec2(), _SPEC_ND, _SPEC_BB],
    out_specs=[_SPEC_ND, _SPEC_ND],
    out_shape=[_SHAPE_ND, _SHAPE_ND],
)


def kernel(X, adj, W0, b0, W1, b1, W2, b2, W3, b3, W4, b4, W5, b5, W6, b6):
    src = adj[0]
    dst = adj[1]
    padW = WCH * CH - E // (NC * NS)
    padC = CCH * CH - E // NS
    padsW = jnp.broadcast_to(N + jnp.arange(padW, dtype=jnp.int32) % NSLACK,
                             (NC * NS, padW))
    padsC = jnp.broadcast_to(N + jnp.arange(padC, dtype=jnp.int32) % NSLACK,
                             (NS, padC))
    srcA = jnp.pad(src.reshape(NC * NS, E // (NC * NS)),
                   ((0, 0), (0, padW))).reshape(NC, NS, WCH * CH // SUB, SUB)
    dstA = jnp.concatenate(
        [dst.reshape(NC * NS, E // (NC * NS)), padsW],
        axis=1).reshape(NC, NS, WCH * CH // SUB, SUB)
    src16 = jnp.pad(src.reshape(NS, E // NS),
                    ((0, 0), (0, padC))).reshape(NS, CCH * CH // SUB, SUB)
    dst16 = jnp.concatenate(
        [dst.reshape(NS, E // NS), padsC],
        axis=1).reshape(NS, CCH * CH // SUB, SUB)
    srcB = jnp.stack([src16, src16 + N])
    dstB = jnp.stack([dst16, dst16])
    onesCD = jnp.ones((CH, D), _f32)
    b0r, b1r, b2r = b0.reshape(1, D), b1.reshape(1, D), b2.reshape(1, D)
    bB35 = jnp.stack([b3.reshape(1, D), b5.reshape(1, D)])
    bB46 = jnp.stack([b4.reshape(1, D), b6.reshape(1, D)])

    srcAd = srcA + (jnp.arange(NC, dtype=jnp.int32) * N)[:, None, None, None]

    def dup(g):
        return jnp.concatenate([g, g])

    degp = _agg_deg(onesCD, srcA, dstA)
    dis, g0 = _mm0(degp, X, W0)
    S0 = _agg_full(dup(g0), srcAd, dstA)
    (g1,) = _mm_mid(S0, g0, dis, b0r, W1)
    S1 = _agg_full(dup(g1), srcAd, dstA)
    (g2,) = _mm_mid(S1, g1, dis, b1r, W2)
    S2 = _agg_full(dup(g2), srcAd, dstA)
    (gB,) = _mm_fork(S2, g2, dis, b2r, W3, W5)
    SB = _agg_branch(gB.reshape(2 * N, D), srcB, dstB)
    (gB2,) = _mm_bmid(SB, gB, dis, bB35, W4, W6)
    SB2 = _agg_branch(gB2.reshape(2 * N, D), srcB, dstB)
    mean, std = _mm_fin(SB2, gB2, dis, bB46)
    return (mean, std)
